# baseline (speedup 1.0000x reference)
import functools

import jax
import jax.numpy as jnp
from jax import lax
from jax.experimental import pallas as pl
from jax.experimental.pallas import tpu as pltpu
from jax.experimental.pallas import tpu_sc as plsc

NUM_CORES = 2
NUM_SUBCORES = 16
CH = 128
FRAC0 = 0.59
BLK = 128


def _sc_scatter_kernel(n_pad, nch0, nch1, nchmax, rows_per_tile):
    mesh = plsc.VectorSubcoreMesh(
        core_axis_name="c", subcore_axis_name="s")

    @functools.partial(
        pl.kernel,
        out_type=jax.ShapeDtypeStruct((NUM_CORES, n_pad, 128), jnp.float32),
        mesh=mesh,
        scratch_types=[
            pltpu.VMEM_SHARED((n_pad, 128), jnp.float32),
            pltpu.VMEM((nchmax, CH), jnp.int32),
            pltpu.VMEM((nchmax, CH), jnp.int32),
            pltpu.VMEM((CH, 128), jnp.float32),
            pltpu.SemaphoreType.DMA,
        ],
    )
    def sc_kernel(h_hbm, src_hbm, dst_hbm, zeros_hbm, m_out,
                  acc_sh, srcw, dstw, rows, sem):
        c = lax.axis_index("c")
        s = lax.axis_index("s")
        wid = c * NUM_SUBCORES + s
        tile_lo = s * rows_per_tile
        pltpu.sync_copy(zeros_hbm, acc_sh.at[pl.ds(tile_lo, rows_per_tile)])
        pltpu.sync_copy(src_hbm.at[wid], srcw)
        pltpu.sync_copy(dst_hbm.at[wid], dstw)
        plsc.subcore_barrier()

        def step(j, carry):
            pltpu.async_copy(h_hbm.at[srcw.at[j]], rows, sem).wait()
            pltpu.sync_copy(rows, acc_sh.at[dstw.at[j]], add=True)
            return carry

        nch_c = jnp.where(c == 0, nch0, nch1)
        lax.fori_loop(0, nch_c, step, 0)
        plsc.subcore_barrier()
        pltpu.sync_copy(acc_sh.at[pl.ds(tile_lo, rows_per_tile)],
                        m_out.at[c, pl.ds(tile_lo, rows_per_tile)])

    return sc_kernel


def _tc_gru(m2, h, hist, wih, whh, bih, bhh, n_valid):
    n_pad, d = h.shape
    grid = n_pad // BLK

    def body(m2_ref, h_ref, hist_ref, wih_ref, whh_ref, bih_ref, bhh_ref,
             hn_ref, histn_ref):
        i = pl.program_id(0)
        m = m2_ref[0] + m2_ref[1]
        hv = h_ref[...]
        gi = jnp.dot(m, wih_ref[...], preferred_element_type=jnp.float32)
        gi = gi + bih_ref[...]
        gh = jnp.dot(hv, whh_ref[...], preferred_element_type=jnp.float32)
        gh = gh + bhh_ref[...]
        r = jax.nn.sigmoid(gi[:, 0:d] + gh[:, 0:d])
        z = jax.nn.sigmoid(gi[:, d:2 * d] + gh[:, d:2 * d])
        n = jnp.tanh(gi[:, 2 * d:3 * d] + r * gh[:, 2 * d:3 * d])
        hn = (1.0 - z) * n + z * hv
        rows = i * BLK + lax.broadcasted_iota(jnp.int32, (BLK, 1), 0)
        hn = jnp.where(rows < n_valid, hn, 0.0)
        hn_ref[...] = hn
        histn_ref[...] = hist_ref[...] + hn

    return pl.pallas_call(
        body,
        grid=(grid,),
        in_specs=[
            pl.BlockSpec((NUM_CORES, BLK, d), lambda i: (0, i, 0)),
            pl.BlockSpec((BLK, d), lambda i: (i, 0)),
            pl.BlockSpec((BLK, d), lambda i: (i, 0)),
            pl.BlockSpec((d, 3 * d), lambda i: (0, 0)),
            pl.BlockSpec((d, 3 * d), lambda i: (0, 0)),
            pl.BlockSpec((1, 3 * d), lambda i: (0, 0)),
            pl.BlockSpec((1, 3 * d), lambda i: (0, 0)),
        ],
        out_specs=[
            pl.BlockSpec((BLK, d), lambda i: (i, 0)),
            pl.BlockSpec((BLK, d), lambda i: (i, 0)),
        ],
        out_shape=[
            jax.ShapeDtypeStruct((n_pad, d), jnp.float32),
            jax.ShapeDtypeStruct((n_pad, d), jnp.float32),
        ],
    )(m2, h, hist, wih, whh, bih, bhh)


def _tc_gru_pool_head(m2, h, hist, wih, whh, bih, bhh, n_valid,
                      batch3, y2, w1, b1, w2, b2, num_graphs):
    n_pad, d = h.shape
    grid = n_pad // BLK
    g = num_graphs

    def body(m2_ref, h_ref, hist_ref, wih_ref, whh_ref, bih_ref, bhh_ref,
             batch_ref, y_ref, w1_ref, b1_ref, w2_ref, b2_ref,
             scores_ref, loss_ref, acc_ref):
        i = pl.program_id(0)

        @pl.when(i == 0)
        def _():
            acc_ref[...] = jnp.zeros_like(acc_ref)

        m = m2_ref[0] + m2_ref[1]
        hv = h_ref[...]
        gi = jnp.dot(m, wih_ref[...], preferred_element_type=jnp.float32)
        gi = gi + bih_ref[...]
        gh = jnp.dot(hv, whh_ref[...], preferred_element_type=jnp.float32)
        gh = gh + bhh_ref[...]
        r = jax.nn.sigmoid(gi[:, 0:d] + gh[:, 0:d])
        z = jax.nn.sigmoid(gi[:, d:2 * d] + gh[:, d:2 * d])
        n = jnp.tanh(gi[:, 2 * d:3 * d] + r * gh[:, 2 * d:3 * d])
        hn = (1.0 - z) * n + z * hv
        rows = i * BLK + lax.broadcasted_iota(jnp.int32, (BLK, 1), 0)
        hn = jnp.where(rows < n_valid, hn, 0.0)
        histn = hist_ref[...] + hn

        b = batch_ref[0]
        gids = lax.broadcasted_iota(jnp.int32, (g, BLK), 0)
        onehot = (b == gids).astype(jnp.float32)
        acc_ref[...] += jnp.dot(onehot, histn,
                                preferred_element_type=jnp.float32)

        @pl.when(i == grid - 1)
        def _():
            pooled = acc_ref[...]
            hdn = jnp.dot(pooled, w1_ref[...],
                          preferred_element_type=jnp.float32) + b1_ref[...]
            hdn = jnp.maximum(hdn, 0.0)
            s = jnp.dot(hdn, w2_ref[...],
                        preferred_element_type=jnp.float32) + b2_ref[...]
            scores_ref[...] = s
            yv = y_ref[...]
            l = jnp.maximum(s, 0.0) - s * yv + jnp.log1p(jnp.exp(-jnp.abs(s)))
            loss_ref[...] = jnp.sum(l, keepdims=True) * (1.0 / g)

    return pl.pallas_call(
        body,
        grid=(grid,),
        in_specs=[
            pl.BlockSpec((NUM_CORES, BLK, d), lambda i: (0, i, 0)),
            pl.BlockSpec((BLK, d), lambda i: (i, 0)),
            pl.BlockSpec((BLK, d), lambda i: (i, 0)),
            pl.BlockSpec((d, 3 * d), lambda i: (0, 0)),
            pl.BlockSpec((d, 3 * d), lambda i: (0, 0)),
            pl.BlockSpec((1, 3 * d), lambda i: (0, 0)),
            pl.BlockSpec((1, 3 * d), lambda i: (0, 0)),
            pl.BlockSpec((1, 1, BLK), lambda i: (i, 0, 0)),
            pl.BlockSpec((g, 1), lambda i: (0, 0)),
            pl.BlockSpec((d, d), lambda i: (0, 0)),
            pl.BlockSpec((1, d), lambda i: (0, 0)),
            pl.BlockSpec((d, 1), lambda i: (0, 0)),
            pl.BlockSpec((1, 1), lambda i: (0, 0)),
        ],
        out_specs=[
            pl.BlockSpec((g, 1), lambda i: (0, 0)),
            pl.BlockSpec((1, 1), lambda i: (0, 0)),
        ],
        out_shape=[
            jax.ShapeDtypeStruct((g, 1), jnp.float32),
            jax.ShapeDtypeStruct((1, 1), jnp.float32),
        ],
        scratch_shapes=[pltpu.VMEM((g, d), jnp.float32)],
    )(m2, h, hist, wih, whh, bih, bhh, batch3, y2, w1, b1, w2, b2)


def kernel(x, edge_index, batch, y, num_graphs, W_ih, W_hh, b_ih, b_hh,
           W1, b1, W2, b2):
    n, d = x.shape
    e = edge_index.shape[1]
    g = y.shape[0]
    num_block = W_ih.shape[0]
    n_workers = NUM_CORES * NUM_SUBCORES

    n_pad = ((n + BLK - 1) // BLK) * BLK
    if n_pad == n:
        n_pad += BLK
    rows_per_tile = n_pad // NUM_SUBCORES
    total_chunks = (e + CH - 1) // CH
    nch0 = -(-int(round(total_chunks * FRAC0)) // NUM_SUBCORES)
    rem = max(total_chunks - nch0 * NUM_SUBCORES, 0)
    nch1 = max(-(-rem // NUM_SUBCORES), 1)
    nchmax = max(nch0, nch1)
    e0 = nch0 * CH * NUM_SUBCORES
    e1 = nch1 * CH * NUM_SUBCORES
    e_pad = e0 + e1

    src = edge_index[0]
    dst = edge_index[1]
    pad_ids = jnp.full((e_pad - e,), n, dtype=jnp.int32)

    def slab(ids):
        idsp = jnp.concatenate([ids, pad_ids])
        a = idsp[:e0].reshape(NUM_SUBCORES, nch0, CH)
        b = idsp[e0:].reshape(NUM_SUBCORES, nch1, CH)
        fill = jnp.full((NUM_SUBCORES, nchmax - nch1, CH), n, jnp.int32)
        b = jnp.concatenate([b, fill], axis=1)
        filla = jnp.full((NUM_SUBCORES, nchmax - nch0, CH), n, jnp.int32)
        a = jnp.concatenate([a, filla], axis=1)
        return jnp.concatenate([a, b], axis=0)

    srcp = slab(src)
    dstp = slab(dst)

    h = jnp.zeros((n_pad, d), jnp.float32).at[:n].set(x)
    hist = h
    zeros_rt = jnp.zeros((rows_per_tile, d), jnp.float32)

    sc_scatter = _sc_scatter_kernel(n_pad, nch0, nch1, nchmax, rows_per_tile)

    bih2 = b_ih.reshape(num_block, 1, 3 * d)
    bhh2 = b_hh.reshape(num_block, 1, 3 * d)
    for i in range(num_block - 1):
        m2 = sc_scatter(h, srcp, dstp, zeros_rt)
        h, hist = _tc_gru(m2, h, hist, W_ih[i], W_hh[i], bih2[i], bhh2[i], n)

    batch3 = jnp.concatenate(
        [batch, jnp.full((n_pad - n,), g, dtype=jnp.int32)]
    ).reshape(n_pad // BLK, 1, BLK)
    m2 = sc_scatter(h, srcp, dstp, zeros_rt)
    i = num_block - 1
    scores2, loss2 = _tc_gru_pool_head(
        m2, h, hist, W_ih[i], W_hh[i], bih2[i], bhh2[i], n,
        batch3, y.reshape(g, 1), W1, b1.reshape(1, d), W2,
        b2.reshape(1, 1), g)
    return scores2[:, 0], loss2[0, 0] + 0.0 * num_graphs


# speedup vs baseline: 1.0409x; 1.0409x over previous
import functools

import jax
import jax.numpy as jnp
from jax import lax
from jax.experimental import pallas as pl
from jax.experimental.pallas import tpu as pltpu
from jax.experimental.pallas import tpu_sc as plsc

NUM_CORES = 2
NUM_SUBCORES = 16
CH = 128
FRAC0 = 0.584
BLK = 128


def _sc_scatter_kernel(n_pad, nch0, nch1, nchmax, rows_per_tile):
    mesh = plsc.VectorSubcoreMesh(
        core_axis_name="c", subcore_axis_name="s")

    @functools.partial(
        pl.kernel,
        out_type=jax.ShapeDtypeStruct((NUM_CORES, n_pad, 128), jnp.float32),
        mesh=mesh,
        scratch_types=[
            pltpu.VMEM_SHARED((n_pad, 128), jnp.float32),
            pltpu.VMEM((nchmax, CH), jnp.int32),
            pltpu.VMEM((nchmax, CH), jnp.int32),
            pltpu.VMEM((CH, 128), jnp.float32),
            pltpu.SemaphoreType.DMA,
        ],
    )
    def sc_kernel(h_hbm, src_hbm, dst_hbm, zeros_hbm, m_out,
                  acc_sh, srcw, dstw, rows, sem):
        c = lax.axis_index("c")
        s = lax.axis_index("s")
        wid = c * NUM_SUBCORES + s
        tile_lo = s * rows_per_tile
        pltpu.sync_copy(zeros_hbm, acc_sh.at[pl.ds(tile_lo, rows_per_tile)])
        pltpu.sync_copy(src_hbm.at[wid], srcw)
        pltpu.sync_copy(dst_hbm.at[wid], dstw)
        plsc.subcore_barrier()

        def step(j, carry):
            pltpu.async_copy(h_hbm.at[srcw.at[j]], rows, sem).wait()
            pltpu.sync_copy(rows, acc_sh.at[dstw.at[j]], add=True)
            return carry

        nch_c = jnp.where(c == 0, nch0, nch1)
        lax.fori_loop(0, nch_c, step, 0)
        plsc.subcore_barrier()
        pltpu.sync_copy(acc_sh.at[pl.ds(tile_lo, rows_per_tile)],
                        m_out.at[c, pl.ds(tile_lo, rows_per_tile)])

    return sc_kernel


def _tc_gru(m2, h, hist, wih, whh, bih, bhh, n_valid):
    n_pad, d = h.shape
    grid = n_pad // BLK

    def body(m2_ref, h_ref, hist_ref, wih_ref, whh_ref, bih_ref, bhh_ref,
             hn_ref, histn_ref):
        i = pl.program_id(0)
        m = m2_ref[0] + m2_ref[1]
        hv = h_ref[...]
        gi = jnp.dot(m, wih_ref[...], preferred_element_type=jnp.float32)
        gi = gi + bih_ref[...]
        gh = jnp.dot(hv, whh_ref[...], preferred_element_type=jnp.float32)
        gh = gh + bhh_ref[...]
        r = jax.nn.sigmoid(gi[:, 0:d] + gh[:, 0:d])
        z = jax.nn.sigmoid(gi[:, d:2 * d] + gh[:, d:2 * d])
        n = jnp.tanh(gi[:, 2 * d:3 * d] + r * gh[:, 2 * d:3 * d])
        hn = (1.0 - z) * n + z * hv
        rows = i * BLK + lax.broadcasted_iota(jnp.int32, (BLK, 1), 0)
        hn = jnp.where(rows < n_valid, hn, 0.0)
        hn_ref[...] = hn
        histn_ref[...] = hist_ref[...] + hn

    return pl.pallas_call(
        body,
        grid=(grid,),
        in_specs=[
            pl.BlockSpec((NUM_CORES, BLK, d), lambda i: (0, i, 0)),
            pl.BlockSpec((BLK, d), lambda i: (i, 0)),
            pl.BlockSpec((BLK, d), lambda i: (i, 0)),
            pl.BlockSpec((d, 3 * d), lambda i: (0, 0)),
            pl.BlockSpec((d, 3 * d), lambda i: (0, 0)),
            pl.BlockSpec((1, 3 * d), lambda i: (0, 0)),
            pl.BlockSpec((1, 3 * d), lambda i: (0, 0)),
        ],
        out_specs=[
            pl.BlockSpec((BLK, d), lambda i: (i, 0)),
            pl.BlockSpec((BLK, d), lambda i: (i, 0)),
        ],
        out_shape=[
            jax.ShapeDtypeStruct((n_pad, d), jnp.float32),
            jax.ShapeDtypeStruct((n_pad, d), jnp.float32),
        ],
    )(m2, h, hist, wih, whh, bih, bhh)


def _tc_gru_pool_head(m2, h, hist, wih, whh, bih, bhh, n_valid,
                      batch3, y2, w1, b1, w2, b2, num_graphs):
    n_pad, d = h.shape
    grid = n_pad // BLK
    g = num_graphs

    def body(m2_ref, h_ref, hist_ref, wih_ref, whh_ref, bih_ref, bhh_ref,
             batch_ref, y_ref, w1_ref, b1_ref, w2_ref, b2_ref,
             scores_ref, loss_ref, acc_ref):
        i = pl.program_id(0)

        @pl.when(i == 0)
        def _():
            acc_ref[...] = jnp.zeros_like(acc_ref)

        m = m2_ref[0] + m2_ref[1]
        hv = h_ref[...]
        gi = jnp.dot(m, wih_ref[...], preferred_element_type=jnp.float32)
        gi = gi + bih_ref[...]
        gh = jnp.dot(hv, whh_ref[...], preferred_element_type=jnp.float32)
        gh = gh + bhh_ref[...]
        r = jax.nn.sigmoid(gi[:, 0:d] + gh[:, 0:d])
        z = jax.nn.sigmoid(gi[:, d:2 * d] + gh[:, d:2 * d])
        n = jnp.tanh(gi[:, 2 * d:3 * d] + r * gh[:, 2 * d:3 * d])
        hn = (1.0 - z) * n + z * hv
        rows = i * BLK + lax.broadcasted_iota(jnp.int32, (BLK, 1), 0)
        hn = jnp.where(rows < n_valid, hn, 0.0)
        histn = hist_ref[...] + hn

        b = batch_ref[0]
        gids = lax.broadcasted_iota(jnp.int32, (g, BLK), 0)
        onehot = (b == gids).astype(jnp.float32)
        acc_ref[...] += jnp.dot(onehot, histn,
                                preferred_element_type=jnp.float32)

        @pl.when(i == grid - 1)
        def _():
            pooled = acc_ref[...]
            hdn = jnp.dot(pooled, w1_ref[...],
                          preferred_element_type=jnp.float32) + b1_ref[...]
            hdn = jnp.maximum(hdn, 0.0)
            s = jnp.dot(hdn, w2_ref[...],
                        preferred_element_type=jnp.float32) + b2_ref[...]
            scores_ref[...] = s
            yv = y_ref[...]
            l = jnp.maximum(s, 0.0) - s * yv + jnp.log1p(jnp.exp(-jnp.abs(s)))
            loss_ref[...] = jnp.sum(l, keepdims=True) * (1.0 / g)

    return pl.pallas_call(
        body,
        grid=(grid,),
        in_specs=[
            pl.BlockSpec((NUM_CORES, BLK, d), lambda i: (0, i, 0)),
            pl.BlockSpec((BLK, d), lambda i: (i, 0)),
            pl.BlockSpec((BLK, d), lambda i: (i, 0)),
            pl.BlockSpec((d, 3 * d), lambda i: (0, 0)),
            pl.BlockSpec((d, 3 * d), lambda i: (0, 0)),
            pl.BlockSpec((1, 3 * d), lambda i: (0, 0)),
            pl.BlockSpec((1, 3 * d), lambda i: (0, 0)),
            pl.BlockSpec((1, 1, BLK), lambda i: (i, 0, 0)),
            pl.BlockSpec((g, 1), lambda i: (0, 0)),
            pl.BlockSpec((d, d), lambda i: (0, 0)),
            pl.BlockSpec((1, d), lambda i: (0, 0)),
            pl.BlockSpec((d, 1), lambda i: (0, 0)),
            pl.BlockSpec((1, 1), lambda i: (0, 0)),
        ],
        out_specs=[
            pl.BlockSpec((g, 1), lambda i: (0, 0)),
            pl.BlockSpec((1, 1), lambda i: (0, 0)),
        ],
        out_shape=[
            jax.ShapeDtypeStruct((g, 1), jnp.float32),
            jax.ShapeDtypeStruct((1, 1), jnp.float32),
        ],
        scratch_shapes=[pltpu.VMEM((g, d), jnp.float32)],
    )(m2, h, hist, wih, whh, bih, bhh, batch3, y2, w1, b1, w2, b2)


def kernel(x, edge_index, batch, y, num_graphs, W_ih, W_hh, b_ih, b_hh,
           W1, b1, W2, b2):
    n, d = x.shape
    e = edge_index.shape[1]
    g = y.shape[0]
    num_block = W_ih.shape[0]
    n_workers = NUM_CORES * NUM_SUBCORES

    n_pad = ((n + BLK - 1) // BLK) * BLK
    if n_pad == n:
        n_pad += BLK
    rows_per_tile = n_pad // NUM_SUBCORES
    total_chunks = (e + CH - 1) // CH
    nch0 = -(-int(round(total_chunks * FRAC0)) // NUM_SUBCORES)
    rem = max(total_chunks - nch0 * NUM_SUBCORES, 0)
    nch1 = max(-(-rem // NUM_SUBCORES), 1)
    nchmax = max(nch0, nch1)
    e0 = nch0 * CH * NUM_SUBCORES
    e1 = nch1 * CH * NUM_SUBCORES
    e_pad = e0 + e1

    src = edge_index[0]
    dst = edge_index[1]
    pad_ids = jnp.full((e_pad - e,), n, dtype=jnp.int32)

    def slab(ids):
        idsp = jnp.concatenate([ids, pad_ids])
        a = idsp[:e0].reshape(NUM_SUBCORES, nch0, CH)
        b = idsp[e0:].reshape(NUM_SUBCORES, nch1, CH)
        fill = jnp.full((NUM_SUBCORES, nchmax - nch1, CH), n, jnp.int32)
        b = jnp.concatenate([b, fill], axis=1)
        filla = jnp.full((NUM_SUBCORES, nchmax - nch0, CH), n, jnp.int32)
        a = jnp.concatenate([a, filla], axis=1)
        return jnp.concatenate([a, b], axis=0)

    srcp = slab(src)
    dstp = slab(dst)

    h = jnp.zeros((n_pad, d), jnp.float32).at[:n].set(x)
    hist = h
    zeros_rt = jnp.zeros((rows_per_tile, d), jnp.float32)

    sc_scatter = _sc_scatter_kernel(n_pad, nch0, nch1, nchmax, rows_per_tile)

    bih2 = b_ih.reshape(num_block, 1, 3 * d)
    bhh2 = b_hh.reshape(num_block, 1, 3 * d)
    for i in range(num_block - 1):
        m2 = sc_scatter(h, srcp, dstp, zeros_rt)
        h, hist = _tc_gru(m2, h, hist, W_ih[i], W_hh[i], bih2[i], bhh2[i], n)

    batch3 = jnp.concatenate(
        [batch, jnp.full((n_pad - n,), g, dtype=jnp.int32)]
    ).reshape(n_pad // BLK, 1, BLK)
    m2 = sc_scatter(h, srcp, dstp, zeros_rt)
    i = num_block - 1
    scores2, loss2 = _tc_gru_pool_head(
        m2, h, hist, W_ih[i], W_hh[i], bih2[i], bhh2[i], n,
        batch3, y.reshape(g, 1), W1, b1.reshape(1, d), W2,
        b2.reshape(1, 1), g)
    return scores2[:, 0], loss2[0, 0] + 0.0 * num_graphs
